# baseline (speedup 1.0000x reference)
import functools

import jax
import jax.numpy as jnp
from jax import lax
from jax.experimental import pallas as pl
from jax.experimental.pallas import tpu as pltpu
from jax.experimental.pallas import tpu_sc as plsc

_B = 16384
_C = 84
_L = 3
_NW = 32
_ROWS = _L * _C
_KMAX = (_ROWS + _NW - 1) // _NW


def _sc_body(s_hbm, neg_hbm, mf_hbm, out_hbm, mask_v, sem):
    nc = 2
    wid = lax.axis_index("s") * nc + lax.axis_index("c")

    pltpu.sync_copy(mf_hbm, mask_v)

    def _fire(k, carry):
        r = wid + k * _NW

        @pl.when(r < _ROWS)
        def _():
            l = r // _C
            c = r - l * _C
            flag = mask_v[pl.ds(r, 16)][0]

            @pl.when(flag == 0)
            def _():
                pltpu.async_copy(s_hbm.at[c], out_hbm.at[l, c], sem)

            @pl.when(flag != 0)
            def _():
                pltpu.async_copy(neg_hbm, out_hbm.at[l, c], sem)

        return carry

    lax.fori_loop(0, _KMAX, _fire, 0)

    def _drain(k, carry):
        r = wid + k * _NW

        @pl.when(r < _ROWS)
        def _():
            l = r // _C
            c = r - l * _C
            pltpu.make_async_copy(neg_hbm, out_hbm.at[l, c], sem).wait()

        return carry

    lax.fori_loop(0, _KMAX, _drain, 0)


def kernel(scores, M):
    B, C = scores.shape
    L = M.shape[0]
    sT = jnp.swapaxes(scores, 0, 1)
    neg_row = jnp.full((B,), -jnp.inf, dtype=jnp.float32)
    mflags = jnp.pad(M.astype(jnp.int32).reshape(L * C), (0, 20))

    mesh = plsc.VectorSubcoreMesh(core_axis_name="c", subcore_axis_name="s")
    k = functools.partial(
        pl.kernel,
        mesh=mesh,
        out_type=jax.ShapeDtypeStruct((L, C, B), jnp.float32),
        scratch_types=[
            pltpu.VMEM((_ROWS + 20,), jnp.int32),
            pltpu.SemaphoreType.DMA,
        ],
    )(_sc_body)
    outT = k(sT, neg_row, mflags)
    return jnp.transpose(outT, (2, 0, 1))


# speedup vs baseline: 16.1664x; 16.1664x over previous
import functools

import jax
import jax.numpy as jnp
from jax import lax
from jax.experimental import pallas as pl
from jax.experimental.pallas import tpu as pltpu
from jax.experimental.pallas import tpu_sc as plsc

_B = 16384
_C = 84
_L = 3
_NW = 32
_ROWS = _L * _C
_KMAX = (_ROWS + _NW - 1) // _NW
_NSLOT = 4


def _sc_body(s_hbm, neg_hbm, mf_hbm, out_hbm, mask_v, fill_v, buf_v,
             sem_fill, sem_g, sem_s):
    nc = 2
    wid = lax.axis_index("s") * nc + lax.axis_index("c")

    pltpu.sync_copy(neg_hbm, fill_v)
    pltpu.sync_copy(mf_hbm, mask_v)

    def _row_lc(k):
        r = wid + k * _NW
        l = r // _C
        c = r - l * _C
        return r, l, c

    def _flag(r):
        return mask_v[pl.ds(r, 16)][0]

    def _fire(k, qg):
        r, l, c = _row_lc(k)
        live = r < _ROWS
        flag = _flag(jnp.where(live, r, 0))
        iscopy = jnp.logical_and(live, flag == 0)
        isfill = jnp.logical_and(live, flag != 0)

        @pl.when(isfill)
        def _():
            pltpu.async_copy(fill_v, out_hbm.at[l, c], sem_fill)

        @pl.when(jnp.logical_and(iscopy, qg < _NSLOT))
        def _():
            pltpu.async_copy(s_hbm.at[c], buf_v.at[qg], sem_g.at[qg])

        return qg + iscopy.astype(jnp.int32)

    nfill_total = lax.fori_loop(0, _KMAX, _fire, jnp.int32(0))
    del nfill_total

    def _pump(k, qc):
        r, l, c = _row_lc(k)
        live = r < _ROWS
        flag = _flag(jnp.where(live, r, 0))
        iscopy = jnp.logical_and(live, flag == 0)
        slot = lax.rem(qc, _NSLOT)

        @pl.when(iscopy)
        def _():
            @pl.when(qc >= _NSLOT)
            def _():
                pltpu.make_async_copy(buf_v.at[slot], out_hbm.at[l, c],
                                      sem_s.at[slot]).wait()
                pltpu.async_copy(s_hbm.at[c], buf_v.at[slot], sem_g.at[slot])

            pltpu.make_async_copy(s_hbm.at[c], buf_v.at[slot],
                                  sem_g.at[slot]).wait()
            pltpu.async_copy(buf_v.at[slot], out_hbm.at[l, c], sem_s.at[slot])

        return qc + iscopy.astype(jnp.int32)

    ncopy = lax.fori_loop(0, _KMAX, _pump, jnp.int32(0))

    def _drain_s(s, carry):
        @pl.when(s < jnp.minimum(ncopy, _NSLOT))
        def _():
            pltpu.make_async_copy(buf_v.at[0], out_hbm.at[0, 0],
                                  sem_s.at[s]).wait()
        return carry

    lax.fori_loop(0, _NSLOT, _drain_s, 0)

    def _drain_f(k, carry):
        r, l, c = _row_lc(k)
        live = r < _ROWS
        flag = _flag(jnp.where(live, r, 0))
        isfill = jnp.logical_and(live, flag != 0)

        @pl.when(isfill)
        def _():
            pltpu.make_async_copy(fill_v, out_hbm.at[l, c], sem_fill).wait()

        return carry

    lax.fori_loop(0, _KMAX, _drain_f, 0)


def kernel(scores, M):
    B, C = scores.shape
    L = M.shape[0]
    sT = jnp.swapaxes(scores, 0, 1)
    neg_row = jnp.full((B,), -jnp.inf, dtype=jnp.float32)
    mflags = jnp.pad(M.astype(jnp.int32).reshape(L * C), (0, 20))

    mesh = plsc.VectorSubcoreMesh(core_axis_name="c", subcore_axis_name="s")
    k = functools.partial(
        pl.kernel,
        mesh=mesh,
        out_type=jax.ShapeDtypeStruct((L, C, B), jnp.float32),
        scratch_types=[
            pltpu.VMEM((_ROWS + 20,), jnp.int32),
            pltpu.VMEM((B,), jnp.float32),
            pltpu.VMEM((_NSLOT, B), jnp.float32),
            pltpu.SemaphoreType.DMA,
            pltpu.SemaphoreType.DMA((_NSLOT,)),
            pltpu.SemaphoreType.DMA((_NSLOT,)),
        ],
    )(_sc_body)
    outT = k(sT, neg_row, mflags)
    return jnp.transpose(outT, (2, 0, 1))


# speedup vs baseline: 54.7812x; 3.3886x over previous
import jax
import jax.numpy as jnp
from jax import lax
from jax.experimental import pallas as pl
from jax.experimental.pallas import tpu as pltpu

_BC = 4096


def _body(s_ref, m_ref, o_hbm, o_v, sems):
    i = pl.program_id(0)
    n = pl.num_programs(0)
    L = m_ref.shape[1]
    slot = lax.rem(i, 2)
    neg = jnp.float32(-jnp.inf)
    s = s_ref[...]

    @pl.when(i >= 2)
    def _():
        for l in range(L):
            pltpu.make_async_copy(
                o_v.at[slot, l],
                o_hbm.at[l, :, pl.ds((i - 2) * _BC, _BC)],
                sems.at[slot, l],
            ).wait()

    for l in range(L):
        ml = m_ref[:, l:l + 1]
        o_v[slot, l] = jnp.where(ml, neg, s)

    for l in range(L):
        pltpu.async_copy(
            o_v.at[slot, l],
            o_hbm.at[l, :, pl.ds(i * _BC, _BC)],
            sems.at[slot, l],
        )

    @pl.when(i == n - 1)
    def _():
        for sl in range(2):
            for l in range(L):
                pltpu.make_async_copy(
                    o_v.at[sl, l],
                    o_hbm.at[l, :, pl.ds(0, _BC)],
                    sems.at[sl, l],
                ).wait()


def kernel(scores, M):
    B, C = scores.shape
    L = M.shape[0]
    sT = jnp.swapaxes(scores, 0, 1)
    mT = jnp.swapaxes(M, 0, 1)
    outT = pl.pallas_call(
        _body,
        grid=(B // _BC,),
        in_specs=[
            pl.BlockSpec((C, _BC), lambda j: (0, j)),
            pl.BlockSpec((C, L), lambda j: (0, 0)),
        ],
        out_specs=pl.BlockSpec(memory_space=pltpu.MemorySpace.HBM),
        out_shape=jax.ShapeDtypeStruct((L, C, B), scores.dtype),
        scratch_shapes=[
            pltpu.VMEM((2, L, C, _BC), jnp.float32),
            pltpu.SemaphoreType.DMA((2, L)),
        ],
    )(sT, mT)
    return jnp.transpose(outT, (2, 0, 1))
